# Initial kernel scaffold; baseline (speedup 1.0000x reference)
import jax
import jax.numpy as jnp
from jax.experimental import pallas as pl


def kernel(idx, table1, table2):
    raise NotImplementedError("write your pallas kernel here")


# speedup vs baseline: 1.0991x; 1.0991x over previous
import functools

import jax
import jax.numpy as jnp
from jax import lax
from jax.experimental import pallas as pl
from jax.experimental.pallas import tpu as pltpu
from jax.experimental.pallas import tpu_sc as plsc

_VOCAB = 1000
_BATCH = 16384
_D = 1000
_NC = 2
_NS = 16
_NW = _NC * _NS
_ROWS_PER_W = _BATCH // _NW
_CHUNK = 64
_NCHUNK = _ROWS_PER_W // _CHUNK
_LANES = 16
_FULL = _D // _LANES
_TAIL_OFF = _D - _LANES


def _add_row_block(buf_a, buf_b, n_rows):
    lane = lax.iota(jnp.int32, _LANES)

    def row_body(i, carry):
        for j in range(_FULL - 1):
            off = j * _LANES
            plsc.addupdate(buf_a.at[i, pl.ds(off, _LANES)],
                           buf_b[i, pl.ds(off, _LANES)])
        off = (_FULL - 1) * _LANES
        plsc.addupdate(buf_a.at[i, pl.ds(off, _LANES)],
                       buf_b[i, pl.ds(off, _LANES)])
        a = buf_a[i, pl.ds(_TAIL_OFF, _LANES)]
        b = buf_b[i, pl.ds(_TAIL_OFF, _LANES)]
        buf_a[i, pl.ds(_TAIL_OFF, _LANES)] = jnp.where(lane >= 8, a + b, a)
        return carry

    lax.fori_loop(0, n_rows, row_body, 0)


def _body(idx0_hbm, idx1_hbm, t1_hbm, t2_hbm, out_hbm,
          idx0_v, idx1_v, buf_a, buf_b, sem_a, sem_b):
    wid = lax.axis_index("s") * _NC + lax.axis_index("c")
    base = wid * _ROWS_PER_W
    pltpu.sync_copy(idx0_hbm.at[wid], idx0_v)
    pltpu.sync_copy(idx1_hbm.at[wid], idx1_v)

    def chunk_body(c, carry):
        cp_a = pltpu.async_copy(t1_hbm.at[idx0_v.at[c]], buf_a, sem_a)
        cp_b = pltpu.async_copy(t2_hbm.at[idx1_v.at[c]], buf_b, sem_b)
        cp_a.wait()
        cp_b.wait()
        _add_row_block(buf_a, buf_b, _CHUNK)
        pltpu.sync_copy(buf_a, out_hbm.at[pl.ds(base + c * _CHUNK, _CHUNK)])
        return carry

    lax.fori_loop(0, _NCHUNK, chunk_body, 0)


@jax.jit
def _sc_bigram(idx0, idx1, table1, table2):
    mesh = plsc.VectorSubcoreMesh(core_axis_name="c", subcore_axis_name="s")
    f = functools.partial(
        pl.kernel,
        out_type=jax.ShapeDtypeStruct((_BATCH, _D), jnp.float32),
        mesh=mesh,
        scratch_types=[
            pltpu.VMEM((_NCHUNK, _CHUNK), jnp.int32),
            pltpu.VMEM((_NCHUNK, _CHUNK), jnp.int32),
            pltpu.VMEM((_CHUNK, _D), jnp.float32),
            pltpu.VMEM((_CHUNK, _D), jnp.float32),
            pltpu.SemaphoreType.DMA,
            pltpu.SemaphoreType.DMA,
        ],
        compiler_params=pltpu.CompilerParams(use_tc_tiling_on_sc=False),
    )(_body)
    return f(idx0, idx1, table1, table2)


def kernel(idx, table1, table2):
    if idx.ndim == 1:
        idx = idx[:, None]
    idx = idx.astype(jnp.int32)
    idx0 = idx[:, 0].reshape(_NW, _NCHUNK, _CHUNK)
    idx1 = idx[:, 1].reshape(_NW, _NCHUNK, _CHUNK)
    return _sc_bigram(idx0, idx1, table1, table2)


# speedup vs baseline: 1.3124x; 1.1941x over previous
import functools

import jax
import jax.numpy as jnp
from jax import lax
from jax.experimental import pallas as pl
from jax.experimental.pallas import tpu as pltpu
from jax.experimental.pallas import tpu_sc as plsc

_VOCAB = 1000
_BATCH = 16384
_D = 1000
_DPAD = 1024
_NC = 2
_NS = 16
_NW = _NC * _NS
_ROWS_PER_W = _BATCH // _NW
_CHUNK = 32
_NCHUNK = _ROWS_PER_W // _CHUNK
_LANES = 16
_FULL = _D // _LANES
_TAIL = _FULL * _LANES


def _sum_rows(buf_a, buf_b, buf_c, n_rows):
    lane = lax.iota(jnp.int32, _LANES)
    tail_mask = lane < (_D - _TAIL)
    tail_col = _TAIL + lane

    def row_body(i, carry):
        for j in range(_FULL):
            off = j * _LANES
            buf_c[i, pl.ds(off, _LANES)] = (buf_a[i, pl.ds(off, _LANES)]
                                            + buf_b[i, pl.ds(off, _LANES)])
        s = buf_a[i, pl.ds(_TAIL, _LANES)] + buf_b[i, pl.ds(_TAIL, _LANES)]
        row = jnp.full((_LANES,), i, dtype=jnp.int32)
        plsc.store_scatter(buf_c, [row, tail_col], s, mask=tail_mask)
        return carry

    lax.fori_loop(0, n_rows, row_body, 0)


def _body(idx0_hbm, idx1_hbm, t1_hbm, t2_hbm, out_hbm,
          idx0_v, idx1_v, buf_a, buf_b, buf_c, sem_a, sem_b):
    wid = lax.axis_index("s") * _NC + lax.axis_index("c")
    base = wid * _ROWS_PER_W
    pltpu.sync_copy(idx0_hbm.at[wid], idx0_v)
    pltpu.sync_copy(idx1_hbm.at[wid], idx1_v)

    def chunk_body(c, carry):
        cp_a = pltpu.async_copy(t1_hbm.at[idx0_v.at[c]], buf_a, sem_a)
        cp_b = pltpu.async_copy(t2_hbm.at[idx1_v.at[c]], buf_b, sem_b)
        cp_a.wait()
        cp_b.wait()
        _sum_rows(buf_a, buf_b, buf_c, _CHUNK)
        pltpu.sync_copy(buf_c, out_hbm.at[pl.ds(base + c * _CHUNK, _CHUNK)])
        return carry

    lax.fori_loop(0, _NCHUNK, chunk_body, 0)


@jax.jit
def _sc_bigram(idx0, idx1, table1, table2):
    mesh = plsc.VectorSubcoreMesh(core_axis_name="c", subcore_axis_name="s")
    f = functools.partial(
        pl.kernel,
        out_type=jax.ShapeDtypeStruct((_BATCH, _D), jnp.float32),
        mesh=mesh,
        scratch_types=[
            pltpu.VMEM((_NCHUNK, _CHUNK), jnp.int32),
            pltpu.VMEM((_NCHUNK, _CHUNK), jnp.int32),
            pltpu.VMEM((_CHUNK, _DPAD), jnp.float32),
            pltpu.VMEM((_CHUNK, _DPAD), jnp.float32),
            pltpu.VMEM((_CHUNK, _D), jnp.float32),
            pltpu.SemaphoreType.DMA,
            pltpu.SemaphoreType.DMA,
        ],
        compiler_params=pltpu.CompilerParams(needs_layout_passes=False),
    )(_body)
    return f(idx0, idx1, table1, table2)


def kernel(idx, table1, table2):
    if idx.ndim == 1:
        idx = idx[:, None]
    idx = idx.astype(jnp.int32)
    idx0 = idx[:, 0].reshape(_NW, _NCHUNK, _CHUNK)
    idx1 = idx[:, 1].reshape(_NW, _NCHUNK, _CHUNK)
    pad = ((0, 0), (0, _DPAD - _D))
    t1p = jnp.pad(table1, pad)
    t2p = jnp.pad(table2, pad)
    return _sc_bigram(idx0, idx1, t1p, t2p)


# speedup vs baseline: 1.7513x; 1.3344x over previous
import functools

import jax
import jax.numpy as jnp
from jax import lax
from jax.experimental import pallas as pl
from jax.experimental.pallas import tpu as pltpu
from jax.experimental.pallas import tpu_sc as plsc

_VOCAB = 1000
_BATCH = 16384
_D = 1000
_DPAD = 1024
_NC = 2
_NS = 16
_NW = _NC * _NS
_ROWS_PER_W = _BATCH // _NW
_CHUNK = 16
_NCHUNK = _ROWS_PER_W // _CHUNK
_LANES = 16
_FULL = _D // _LANES
_TAIL = _FULL * _LANES


def _sum_rows(buf_a, buf_b, buf_c, n_rows):
    lane = lax.iota(jnp.int32, _LANES)
    tail_mask = lane < (_D - _TAIL)
    tail_col = _TAIL + lane

    def row_body(i, carry):
        for j in range(_FULL):
            off = j * _LANES
            buf_c[i, pl.ds(off, _LANES)] = (buf_a[i, pl.ds(off, _LANES)]
                                            + buf_b[i, pl.ds(off, _LANES)])
        s = buf_a[i, pl.ds(_TAIL, _LANES)] + buf_b[i, pl.ds(_TAIL, _LANES)]
        row = jnp.full((_LANES,), i, dtype=jnp.int32)
        plsc.store_scatter(buf_c, [row, tail_col], s, mask=tail_mask)
        return carry

    lax.fori_loop(0, n_rows, row_body, 0)


def _body(idx0_hbm, idx1_hbm, t1_hbm, t2_hbm, out_hbm,
          idx0_v, idx1_v,
          buf_a0, buf_b0, buf_a1, buf_b1, buf_c0, buf_c1,
          sem_a0, sem_b0, sem_a1, sem_b1, sem_w0, sem_w1):
    wid = lax.axis_index("s") * _NC + lax.axis_index("c")
    base = wid * _ROWS_PER_W
    pltpu.sync_copy(idx0_hbm.at[wid], idx0_v)
    pltpu.sync_copy(idx1_hbm.at[wid], idx1_v)

    def gather(c, buf_a, buf_b, sem_a, sem_b):
        pltpu.async_copy(t1_hbm.at[idx0_v.at[c]], buf_a, sem_a)
        pltpu.async_copy(t2_hbm.at[idx1_v.at[c]], buf_b, sem_b)

    def wait_gather(c, buf_a, buf_b, sem_a, sem_b):
        pltpu.make_async_copy(t1_hbm.at[idx0_v.at[c]], buf_a, sem_a).wait()
        pltpu.make_async_copy(t2_hbm.at[idx1_v.at[c]], buf_b, sem_b).wait()

    def out_slice(c):
        return out_hbm.at[pl.ds(base + c * _CHUNK, _CHUNK)]

    gather(0, buf_a0, buf_b0, sem_a0, sem_b0)

    def pair_body(k, carry):
        c0 = 2 * k
        c1 = 2 * k + 1
        gather(c1, buf_a1, buf_b1, sem_a1, sem_b1)
        wait_gather(c0, buf_a0, buf_b0, sem_a0, sem_b0)

        @pl.when(k > 0)
        def _():
            pltpu.make_async_copy(buf_c0, out_slice(c0), sem_w0).wait()
        _sum_rows(buf_a0, buf_b0, buf_c0, _CHUNK)
        pltpu.async_copy(buf_c0, out_slice(c0), sem_w0)

        gather(jnp.minimum(c0 + 2, _NCHUNK - 1), buf_a0, buf_b0,
               sem_a0, sem_b0)
        wait_gather(c1, buf_a1, buf_b1, sem_a1, sem_b1)

        @pl.when(k > 0)
        def _():
            pltpu.make_async_copy(buf_c1, out_slice(c1), sem_w1).wait()
        _sum_rows(buf_a1, buf_b1, buf_c1, _CHUNK)
        pltpu.async_copy(buf_c1, out_slice(c1), sem_w1)
        return carry

    lax.fori_loop(0, _NCHUNK // 2, pair_body, 0)
    wait_gather(_NCHUNK - 1, buf_a0, buf_b0, sem_a0, sem_b0)
    pltpu.make_async_copy(buf_c0, out_slice(_NCHUNK - 2), sem_w0).wait()
    pltpu.make_async_copy(buf_c1, out_slice(_NCHUNK - 1), sem_w1).wait()


@jax.jit
def _sc_bigram(idx0, idx1, table1, table2):
    mesh = plsc.VectorSubcoreMesh(core_axis_name="c", subcore_axis_name="s")
    f = functools.partial(
        pl.kernel,
        out_type=jax.ShapeDtypeStruct((_BATCH, _D), jnp.float32),
        mesh=mesh,
        scratch_types=[
            pltpu.VMEM((_NCHUNK, _CHUNK), jnp.int32),
            pltpu.VMEM((_NCHUNK, _CHUNK), jnp.int32),
            pltpu.VMEM((_CHUNK, _DPAD), jnp.float32),
            pltpu.VMEM((_CHUNK, _DPAD), jnp.float32),
            pltpu.VMEM((_CHUNK, _DPAD), jnp.float32),
            pltpu.VMEM((_CHUNK, _DPAD), jnp.float32),
            pltpu.VMEM((_CHUNK, _D), jnp.float32),
            pltpu.VMEM((_CHUNK, _D), jnp.float32),
            pltpu.SemaphoreType.DMA,
            pltpu.SemaphoreType.DMA,
            pltpu.SemaphoreType.DMA,
            pltpu.SemaphoreType.DMA,
            pltpu.SemaphoreType.DMA,
            pltpu.SemaphoreType.DMA,
        ],
        compiler_params=pltpu.CompilerParams(needs_layout_passes=False),
    )(_body)
    return f(idx0, idx1, table1, table2)


def kernel(idx, table1, table2):
    if idx.ndim == 1:
        idx = idx[:, None]
    idx = idx.astype(jnp.int32)
    idx0 = idx[:, 0].reshape(_NW, _NCHUNK, _CHUNK)
    idx1 = idx[:, 1].reshape(_NW, _NCHUNK, _CHUNK)
    pad = ((0, 0), (0, _DPAD - _D))
    t1p = jnp.pad(table1, pad)
    t2p = jnp.pad(table2, pad)
    return _sc_bigram(idx0, idx1, t1p, t2p)
